# baseline (speedup 1.0000x reference)
import jax
import jax.numpy as jnp
from jax import lax
from jax.experimental import pallas as pl
from jax.experimental.pallas import tpu as pltpu

_BATCH = 4096
_N_FIELDS = 26
_CARD = 200
_OUT_COLS = _N_FIELDS * _CARD
_LANES = 128
_NWIN = (_OUT_COLS + _LANES - 1) // _LANES
_R = 512


def _body(x_ref, o_ref):
    y = x_ref[...] + _CARD * lax.broadcasted_iota(jnp.int32, (1, _N_FIELDS), 1)
    for j in range(_NWIN):
        lo = j * _LANES
        width = min(_LANES, _OUT_COLS - lo)
        i0 = lo // _CARD
        i1 = min(_N_FIELDS - 1, (lo + width - 1) // _CARD)
        c = lo + lax.broadcasted_iota(jnp.int32, (_R, width), 1)
        m = y[:, i0:i0 + 1] == c
        if i1 != i0:
            m = m | (y[:, i1:i1 + 1] == c)
        o_ref[:, lo:lo + width] = m.astype(jnp.int8)


@jax.jit
def _onehot_tc(x):
    out8 = pl.pallas_call(
        _body,
        grid=(_BATCH // _R,),
        in_specs=[pl.BlockSpec((_R, _N_FIELDS), lambda i: (i, 0))],
        out_specs=pl.BlockSpec((_R, _OUT_COLS), lambda i: (i, 0)),
        out_shape=jax.ShapeDtypeStruct((_BATCH, _OUT_COLS), jnp.int8),
        compiler_params=pltpu.CompilerParams(
            dimension_semantics=("arbitrary",)),
    )(x)
    return out8.astype(jnp.int32)


def kernel(x):
    return _onehot_tc(x)
